# Initial kernel scaffold; baseline (speedup 1.0000x reference)
import jax
import jax.numpy as jnp
from jax.experimental import pallas as pl


def kernel(z1, x_pos, params, edge_index):
    raise NotImplementedError("write your pallas kernel here")


# speedup vs baseline: 4.5344x; 4.5344x over previous
import functools

import jax
import jax.numpy as jnp
from jax import lax
from jax.experimental import pallas as pl
from jax.experimental.pallas import tpu as pltpu
from jax.experimental.pallas import tpu_sc as plsc

N_REAL = 100000
N_PAD = 102400
E_TOT = 3200000
N_TILES = 16

SUB = 400
SUBS = 4
SUPER = SUB * SUBS
N_SUPER = E_TOT // SUPER
SUPERS_PER_TILE = N_SUPER // N_TILES
ROWS_PER_TILE = N_PAD // N_TILES

DEG_BATCH = 800
DEG_EDGES_PER_TILE = E_TOT // 32
DEG_N_BATCHES = DEG_EDGES_PER_TILE // DEG_BATCH

BN_EPS = 1e-5
MM_BLOCK = 1024


def _sc_propagate(h_cm, src3, dst3):
    mesh = plsc.VectorSubcoreMesh(core_axis_name="c", subcore_axis_name="s")
    n_chunks = h_cm.shape[0]

    @functools.partial(
        pl.kernel,
        out_type=jax.ShapeDtypeStruct((n_chunks, N_PAD, 16), jnp.float32),
        mesh=mesh,
        compiler_params=pltpu.CompilerParams(use_tc_tiling_on_sc=False),
        scratch_types=[
            pltpu.VMEM_SHARED((N_PAD, 16), jnp.float32),
            pltpu.VMEM((2, SUBS, SUB), jnp.int32),
            pltpu.VMEM((2, SUBS, SUB), jnp.int32),
            pltpu.VMEM((2, SUB, 16), jnp.float32),
            pltpu.SemaphoreType.DMA,
            pltpu.SemaphoreType.DMA,
            pltpu.SemaphoreType.DMA,
            pltpu.SemaphoreType.DMA,
        ],
    )
    def prop(h_hbm, src_hbm, dst_hbm, out_hbm, acc_sh, src_v, dst_v, rows_v,
             sem_g0, sem_g1, sem_s0, sem_s1):
        cid = lax.axis_index("c")
        sid = lax.axis_index("s")
        sems = (sem_s0, sem_s1)
        gsems = (sem_g0, sem_g1)

        def wait_scatter(p):
            pltpu.make_async_copy(
                rows_v.at[p], acc_sh.at[pl.ds(0, SUB)], sems[p]).wait()

        def chunk_body(k):
            def zfill(i, _):
                rows_v[0, i, :] = jnp.zeros((16,), jnp.float32)
                return 0
            lax.fori_loop(0, SUB, zfill, 0)

            def zero_step(j, _):
                r0 = sid * ROWS_PER_TILE + j * SUB
                pltpu.sync_copy(rows_v.at[0], acc_sh.at[pl.ds(r0, SUB)])
                return 0
            lax.fori_loop(0, ROWS_PER_TILE // SUB, zero_step, 0)
            plsc.subcore_barrier()

            def super_step(s, _):
                sp = s % 2
                sv = src_v.at[sp]
                dv = dst_v.at[sp]
                r = sid * SUPERS_PER_TILE + s
                pltpu.sync_copy(src_hbm.at[r], sv)
                pltpu.sync_copy(dst_hbm.at[r], dv)

                def gwait(j):
                    q = j % 2
                    pltpu.make_async_copy(
                        h_hbm.at[k].at[sv.at[j]], rows_v.at[q],
                        gsems[q]).wait()

                def scat(j):
                    q = j % 2
                    pltpu.async_copy(rows_v.at[q], acc_sh.at[dv.at[j]],
                                     sems[q], add=True)

                for j in range(SUBS):
                    p = j % 2

                    @pl.when((s > 0) | (j >= 2))
                    def _():
                        wait_scatter(p)
                    pltpu.async_copy(h_hbm.at[k].at[sv.at[j]],
                                     rows_v.at[p], gsems[p])
                    if j >= 1:
                        gwait(j - 1)
                        scat(j - 1)
                gwait(SUBS - 1)
                scat(SUBS - 1)
                return 0
            lax.fori_loop(0, SUPERS_PER_TILE, super_step, 0)
            wait_scatter(0)
            wait_scatter(1)
            plsc.subcore_barrier()

            r0 = sid * ROWS_PER_TILE
            pltpu.sync_copy(acc_sh.at[pl.ds(r0, ROWS_PER_TILE)],
                            out_hbm.at[k, pl.ds(r0, ROWS_PER_TILE)])
            plsc.subcore_barrier()

        if n_chunks == 1:
            @pl.when(cid == 0)
            def _():
                chunk_body(0)
        else:
            def chunk_step(i, _):
                chunk_body(2 * i + cid)
                return 0
            lax.fori_loop(0, n_chunks // 2, chunk_step, 0)

    return prop(h_cm, src3, dst3)


def _to_cm(h):
    K = h.shape[1] // 16
    if K == 1:
        return h.reshape(1, N_PAD, 16)
    return h.reshape(N_PAD, K, 16).transpose(1, 0, 2)


def _from_cm(s_cm):
    K = s_cm.shape[0]
    if K == 1:
        return s_cm.reshape(N_PAD, 16)
    return s_cm.transpose(1, 0, 2).reshape(N_PAD, K * 16)


def _sc_degree(dst):
    mesh = plsc.VectorSubcoreMesh(core_axis_name="c", subcore_axis_name="s")

    @functools.partial(
        pl.kernel,
        out_type=jax.ShapeDtypeStruct((2, N_PAD), jnp.float32),
        mesh=mesh,
        compiler_params=pltpu.CompilerParams(use_tc_tiling_on_sc=False),
        scratch_types=[
            pltpu.VMEM_SHARED((N_PAD,), jnp.float32),
            pltpu.VMEM((DEG_BATCH,), jnp.int32),
            pltpu.VMEM((DEG_BATCH,), jnp.float32),
        ],
    )
    def degk(dst_hbm, out_hbm, acc_sh, dst_v, val_v):
        cid = lax.axis_index("c")
        sid = lax.axis_index("s")

        def zfill(i, _):
            val_v[pl.ds(16 * i, 16)] = jnp.zeros((16,), jnp.float32)
            return 0
        lax.fori_loop(0, DEG_BATCH // 16, zfill, 0)

        def zero_step(j, _):
            r0 = sid * ROWS_PER_TILE + j * DEG_BATCH
            pltpu.sync_copy(val_v, acc_sh.at[pl.ds(r0, DEG_BATCH)])
            return 0
        lax.fori_loop(0, ROWS_PER_TILE // DEG_BATCH, zero_step, 0)
        plsc.subcore_barrier()

        def ofill(i, _):
            val_v[pl.ds(16 * i, 16)] = jnp.ones((16,), jnp.float32)
            return 0
        lax.fori_loop(0, DEG_BATCH // 16, ofill, 0)

        def edge_step(b, _):
            e0 = (cid * 16 + sid) * DEG_EDGES_PER_TILE + b * DEG_BATCH
            pltpu.sync_copy(dst_hbm.at[pl.ds(e0, DEG_BATCH)], dst_v)
            pltpu.sync_copy(val_v, acc_sh.at[dst_v], add=True)
            return 0
        lax.fori_loop(0, DEG_N_BATCHES, edge_step, 0)
        plsc.subcore_barrier()

        def drain_step(j, _):
            r0 = sid * ROWS_PER_TILE + j * DEG_BATCH
            pltpu.sync_copy(acc_sh.at[pl.ds(r0, DEG_BATCH)], val_v)
            pltpu.sync_copy(val_v, out_hbm.at[cid, pl.ds(r0, DEG_BATCH)])
            return 0
        lax.fori_loop(0, ROWS_PER_TILE // DEG_BATCH, drain_step, 0)

    return degk(dst)


def _grid(n):
    return (n // MM_BLOCK,)


def _tc_dinv(deg2):
    def body(d_ref, o_ref):
        deg = d_ref[0, :] + d_ref[1, :] + 1.0
        o_ref[...] = (1.0 / jnp.sqrt(jnp.maximum(deg, 1.0)))[:, None]

    return pl.pallas_call(
        body,
        grid=_grid(N_PAD),
        in_specs=[pl.BlockSpec((2, MM_BLOCK), lambda i: (0, i))],
        out_specs=pl.BlockSpec((MM_BLOCK, 1), lambda i: (i, 0)),
        out_shape=jax.ShapeDtypeStruct((N_PAD, 1), jnp.float32),
    )(deg2)


def _tc_scale(x, dinv):
    F = x.shape[1]

    def body(x_ref, d_ref, o_ref):
        o_ref[...] = x_ref[...] * d_ref[...]

    return pl.pallas_call(
        body,
        grid=_grid(N_PAD),
        in_specs=[pl.BlockSpec((MM_BLOCK, F), lambda i: (i, 0)),
                  pl.BlockSpec((MM_BLOCK, 1), lambda i: (i, 0))],
        out_specs=pl.BlockSpec((MM_BLOCK, F), lambda i: (i, 0)),
        out_shape=jax.ShapeDtypeStruct((N_PAD, F), jnp.float32),
    )(x, dinv)


def _tc_mm_scale(x, w, dinv):
    fin, fout = w.shape

    def body(x_ref, w_ref, d_ref, o_ref):
        y = jnp.dot(x_ref[...], w_ref[...],
                    preferred_element_type=jnp.float32)
        o_ref[...] = y * d_ref[...]

    return pl.pallas_call(
        body,
        grid=_grid(N_PAD),
        in_specs=[pl.BlockSpec((MM_BLOCK, fin), lambda i: (i, 0)),
                  pl.BlockSpec((fin, fout), lambda i: (0, 0)),
                  pl.BlockSpec((MM_BLOCK, 1), lambda i: (i, 0))],
        out_specs=pl.BlockSpec((MM_BLOCK, fout), lambda i: (i, 0)),
        out_shape=jax.ShapeDtypeStruct((N_PAD, fout), jnp.float32),
    )(x, w, dinv)


def _tc_comb(s, ms, dinv):
    F = s.shape[1]

    def body(s_ref, m_ref, d_ref, o_ref, s1_ref, s2_ref):
        p = (s_ref[...] + m_ref[...]) * d_ref[...]
        o_ref[...] = p

        @pl.when(pl.program_id(0) == 0)
        def _():
            s1_ref[...] = jnp.zeros_like(s1_ref)
            s2_ref[...] = jnp.zeros_like(s2_ref)
        s1_ref[...] += jnp.sum(p, axis=0, keepdims=True)
        s2_ref[...] += jnp.sum(p * p, axis=0, keepdims=True)

    return pl.pallas_call(
        body,
        grid=_grid(N_PAD),
        in_specs=[pl.BlockSpec((MM_BLOCK, F), lambda i: (i, 0)),
                  pl.BlockSpec((MM_BLOCK, F), lambda i: (i, 0)),
                  pl.BlockSpec((MM_BLOCK, 1), lambda i: (i, 0))],
        out_specs=[pl.BlockSpec((MM_BLOCK, F), lambda i: (i, 0)),
                   pl.BlockSpec((1, F), lambda i: (0, 0)),
                   pl.BlockSpec((1, F), lambda i: (0, 0))],
        out_shape=[jax.ShapeDtypeStruct((N_PAD, F), jnp.float32),
                   jax.ShapeDtypeStruct((1, F), jnp.float32),
                   jax.ShapeDtypeStruct((1, F), jnp.float32)],
    )(s, ms, dinv)


def _tc_mm_comb(s, xs, dinv, w):
    fin, fout = w.shape

    def body(s_ref, x_ref, d_ref, w_ref, o_ref, s1_ref, s2_ref):
        g = (s_ref[...] + x_ref[...]) * d_ref[...]
        p = jnp.dot(g, w_ref[...], preferred_element_type=jnp.float32)
        o_ref[...] = p

        @pl.when(pl.program_id(0) == 0)
        def _():
            s1_ref[...] = jnp.zeros_like(s1_ref)
            s2_ref[...] = jnp.zeros_like(s2_ref)
        s1_ref[...] += jnp.sum(p, axis=0, keepdims=True)
        s2_ref[...] += jnp.sum(p * p, axis=0, keepdims=True)

    return pl.pallas_call(
        body,
        grid=_grid(N_PAD),
        in_specs=[pl.BlockSpec((MM_BLOCK, fin), lambda i: (i, 0)),
                  pl.BlockSpec((MM_BLOCK, fin), lambda i: (i, 0)),
                  pl.BlockSpec((MM_BLOCK, 1), lambda i: (i, 0)),
                  pl.BlockSpec((fin, fout), lambda i: (0, 0))],
        out_specs=[pl.BlockSpec((MM_BLOCK, fout), lambda i: (i, 0)),
                   pl.BlockSpec((1, fout), lambda i: (0, 0)),
                   pl.BlockSpec((1, fout), lambda i: (0, 0))],
        out_shape=[jax.ShapeDtypeStruct((N_PAD, fout), jnp.float32),
                   jax.ShapeDtypeStruct((1, fout), jnp.float32),
                   jax.ShapeDtypeStruct((1, fout), jnp.float32)],
    )(s, xs, dinv, w)


def _tc_act(p, s1, s2, gamma, beta, dinv, scale_out):
    F = p.shape[1]

    def body(p_ref, s1_ref, s2_ref, g_ref, b_ref, d_ref, o_ref):
        inv_n = jnp.float32(1.0 / N_REAL)
        mean = s1_ref[...] * inv_n
        var = s2_ref[...] * inv_n - mean * mean
        rstd = 1.0 / jnp.sqrt(var + BN_EPS)
        y = (p_ref[...] - mean) * (rstd * g_ref[...]) + b_ref[...]
        y = jnp.where(y > 0, y, 0.01 * y)
        if scale_out:
            y = y * d_ref[...]
        row = (pl.program_id(0) * MM_BLOCK
               + lax.broadcasted_iota(jnp.int32, (MM_BLOCK, 1), 0))
        o_ref[...] = jnp.where(row < N_REAL, y, 0.0)

    return pl.pallas_call(
        body,
        grid=_grid(N_PAD),
        in_specs=[pl.BlockSpec((MM_BLOCK, F), lambda i: (i, 0)),
                  pl.BlockSpec((1, F), lambda i: (0, 0)),
                  pl.BlockSpec((1, F), lambda i: (0, 0)),
                  pl.BlockSpec((1, F), lambda i: (0, 0)),
                  pl.BlockSpec((1, F), lambda i: (0, 0)),
                  pl.BlockSpec((MM_BLOCK, 1), lambda i: (i, 0))],
        out_specs=pl.BlockSpec((MM_BLOCK, F), lambda i: (i, 0)),
        out_shape=jax.ShapeDtypeStruct((N_PAD, F), jnp.float32),
    )(p, s1, s2, gamma.reshape(1, F), beta.reshape(1, F), dinv)


def _tc_head(x, w1, b1, w2, b2, xpos):
    fin, fmid = w1.shape
    fout = w2.shape[1]

    def body(x_ref, w1_ref, b1_ref, w2_ref, b2_ref, xp_ref, o_ref):
        h = jnp.dot(x_ref[...], w1_ref[...],
                    preferred_element_type=jnp.float32) + b1_ref[...]
        h = jnp.where(h > 0, h, 0.01 * h)
        o = jnp.dot(h, w2_ref[...],
                    preferred_element_type=jnp.float32) + b2_ref[...]
        o_ref[...] = o + xp_ref[...]

    return pl.pallas_call(
        body,
        grid=_grid(N_PAD),
        in_specs=[pl.BlockSpec((MM_BLOCK, fin), lambda i: (i, 0)),
                  pl.BlockSpec((fin, fmid), lambda i: (0, 0)),
                  pl.BlockSpec((1, fmid), lambda i: (0, 0)),
                  pl.BlockSpec((fmid, fout), lambda i: (0, 0)),
                  pl.BlockSpec((1, fout), lambda i: (0, 0)),
                  pl.BlockSpec((MM_BLOCK, fout), lambda i: (i, 0))],
        out_specs=pl.BlockSpec((MM_BLOCK, fout), lambda i: (i, 0)),
        out_shape=jax.ShapeDtypeStruct((N_PAD, fout), jnp.float32),
    )(x, w1, b1.reshape(1, fmid), w2, b2.reshape(1, fout), xpos)


def kernel(z1, x_pos, params, edge_index):
    n = z1.shape[0]
    src3 = edge_index[0].astype(jnp.int32).reshape(N_SUPER, SUBS, SUB)
    dst = edge_index[1].astype(jnp.int32)
    dst3 = dst.reshape(N_SUPER, SUBS, SUB)
    xpad = jnp.pad(z1, ((0, N_PAD - n), (0, 0)))
    xpos_pad = jnp.pad(x_pos, ((0, N_PAD - n), (0, 0)))

    deg2 = _sc_degree(dst)
    dinv = _tc_dinv(deg2)

    hdims = [16, 32, 64, 128, 256, 256, 512, 512, 256, 256, 128, 64, 32]
    prop_first = [False] * 12

    x = xpad
    for i in range(12):
        w, _, gamma, beta = params[i]
        fin, fout = hdims[i], hdims[i + 1]
        if prop_first[i]:
            s = _from_cm(_sc_propagate(_to_cm(x), src3, dst3))
            p, s1, s2 = _tc_mm_comb(s, x, dinv, w)
        else:
            ms = _tc_mm_scale(x, w, dinv)
            s = _from_cm(_sc_propagate(_to_cm(ms), src3, dst3))
            p, s1, s2 = _tc_comb(s, ms, dinv)
        scale_next = prop_first[i + 1] if i < 11 else False
        x = _tc_act(p, s1, s2, gamma, beta, dinv, scale_next)

    w1, b1 = params[12]
    w2, b2 = params[13]
    out = _tc_head(x, w1, b1, w2, b2, xpos_pad)
    return out[:n]


# speedup vs baseline: 4.6340x; 1.0220x over previous
import functools

import jax
import jax.numpy as jnp
from jax import lax
from jax.experimental import pallas as pl
from jax.experimental.pallas import tpu as pltpu
from jax.experimental.pallas import tpu_sc as plsc

N_REAL = 100000
N_PAD = 102400
E_TOT = 3200000
N_TILES = 16

SUB = 400
SUBS = 4
SUPER = SUB * SUBS
N_SUPER = E_TOT // SUPER
SUPERS_PER_TILE = N_SUPER // N_TILES
ROWS_PER_TILE = N_PAD // N_TILES

DEG_BATCH = 800
DEG_EDGES_PER_TILE = E_TOT // 32
DEG_N_BATCHES = DEG_EDGES_PER_TILE // DEG_BATCH

BN_EPS = 1e-5
MM_BLOCK = 1024


def _sc_propagate(h_cm, src3, dst3):
    mesh = plsc.VectorSubcoreMesh(core_axis_name="c", subcore_axis_name="s")
    n_chunks = h_cm.shape[0]

    @functools.partial(
        pl.kernel,
        out_type=jax.ShapeDtypeStruct((n_chunks, N_PAD, 16), jnp.float32),
        mesh=mesh,
        compiler_params=pltpu.CompilerParams(use_tc_tiling_on_sc=False),
        scratch_types=[
            pltpu.VMEM_SHARED((N_PAD, 16), jnp.float32),
            pltpu.VMEM((2, SUBS, SUB), jnp.int32),
            pltpu.VMEM((2, SUBS, SUB), jnp.int32),
            pltpu.VMEM((2, SUB, 16), jnp.float32),
            pltpu.SemaphoreType.DMA,
            pltpu.SemaphoreType.DMA,
            pltpu.SemaphoreType.DMA,
            pltpu.SemaphoreType.DMA,
        ],
    )
    def prop(h_hbm, src_hbm, dst_hbm, out_hbm, acc_sh, src_v, dst_v, rows_v,
             sem_g0, sem_g1, sem_s0, sem_s1):
        cid = lax.axis_index("c")
        sid = lax.axis_index("s")
        sems = (sem_s0, sem_s1)
        gsems = (sem_g0, sem_g1)

        def wait_scatter(p):
            pltpu.make_async_copy(
                rows_v.at[p], acc_sh.at[pl.ds(0, SUB)], sems[p]).wait()

        def chunk_body(k):
            def zfill(i, _):
                rows_v[0, i, :] = jnp.zeros((16,), jnp.float32)
                return 0
            lax.fori_loop(0, SUB, zfill, 0)

            def zero_step(j, _):
                r0 = sid * ROWS_PER_TILE + j * SUB
                pltpu.sync_copy(rows_v.at[0], acc_sh.at[pl.ds(r0, SUB)])
                return 0
            lax.fori_loop(0, ROWS_PER_TILE // SUB, zero_step, 0)
            plsc.subcore_barrier()

            def super_step(s, _):
                sp = s % 2
                sv = src_v.at[sp]
                dv = dst_v.at[sp]
                r = sid * SUPERS_PER_TILE + s
                pltpu.sync_copy(src_hbm.at[r], sv)
                pltpu.sync_copy(dst_hbm.at[r], dv)

                def gwait(j):
                    q = j % 2
                    pltpu.make_async_copy(
                        h_hbm.at[k].at[sv.at[j]], rows_v.at[q],
                        gsems[q]).wait()

                def scat(j):
                    q = j % 2
                    pltpu.async_copy(rows_v.at[q], acc_sh.at[dv.at[j]],
                                     sems[q], add=True)

                for j in range(SUBS):
                    p = j % 2

                    @pl.when((s > 0) | (j >= 2))
                    def _():
                        wait_scatter(p)
                    pltpu.async_copy(h_hbm.at[k].at[sv.at[j]],
                                     rows_v.at[p], gsems[p])
                    if j >= 1:
                        gwait(j - 1)
                        scat(j - 1)
                gwait(SUBS - 1)
                scat(SUBS - 1)
                return 0
            lax.fori_loop(0, SUPERS_PER_TILE, super_step, 0)
            wait_scatter(0)
            wait_scatter(1)
            plsc.subcore_barrier()

            r0 = sid * ROWS_PER_TILE
            pltpu.sync_copy(acc_sh.at[pl.ds(r0, ROWS_PER_TILE)],
                            out_hbm.at[k, pl.ds(r0, ROWS_PER_TILE)])
            plsc.subcore_barrier()

        if n_chunks == 1:
            @pl.when(cid == 0)
            def _():
                chunk_body(0)
        else:
            def chunk_step(i, _):
                chunk_body(2 * i + cid)
                return 0
            lax.fori_loop(0, n_chunks // 2, chunk_step, 0)

    return prop(h_cm, src3, dst3)


def _to_cm(h):
    K = h.shape[1] // 16
    if K == 1:
        return h.reshape(1, N_PAD, 16)
    return h.reshape(N_PAD, K, 16).transpose(1, 0, 2)


def _from_cm(s_cm):
    K = s_cm.shape[0]
    if K == 1:
        return s_cm.reshape(N_PAD, 16)
    return s_cm.transpose(1, 0, 2).reshape(N_PAD, K * 16)


def _sc_degree(dst):
    mesh = plsc.VectorSubcoreMesh(core_axis_name="c", subcore_axis_name="s")

    @functools.partial(
        pl.kernel,
        out_type=jax.ShapeDtypeStruct((2, N_PAD), jnp.float32),
        mesh=mesh,
        compiler_params=pltpu.CompilerParams(use_tc_tiling_on_sc=False),
        scratch_types=[
            pltpu.VMEM_SHARED((N_PAD,), jnp.float32),
            pltpu.VMEM((DEG_BATCH,), jnp.int32),
            pltpu.VMEM((DEG_BATCH,), jnp.float32),
        ],
    )
    def degk(dst_hbm, out_hbm, acc_sh, dst_v, val_v):
        cid = lax.axis_index("c")
        sid = lax.axis_index("s")

        def zfill(i, _):
            val_v[pl.ds(16 * i, 16)] = jnp.zeros((16,), jnp.float32)
            return 0
        lax.fori_loop(0, DEG_BATCH // 16, zfill, 0)

        def zero_step(j, _):
            r0 = sid * ROWS_PER_TILE + j * DEG_BATCH
            pltpu.sync_copy(val_v, acc_sh.at[pl.ds(r0, DEG_BATCH)])
            return 0
        lax.fori_loop(0, ROWS_PER_TILE // DEG_BATCH, zero_step, 0)
        plsc.subcore_barrier()

        def ofill(i, _):
            val_v[pl.ds(16 * i, 16)] = jnp.ones((16,), jnp.float32)
            return 0
        lax.fori_loop(0, DEG_BATCH // 16, ofill, 0)

        def edge_step(b, _):
            e0 = (cid * 16 + sid) * DEG_EDGES_PER_TILE + b * DEG_BATCH
            pltpu.sync_copy(dst_hbm.at[pl.ds(e0, DEG_BATCH)], dst_v)
            pltpu.sync_copy(val_v, acc_sh.at[dst_v], add=True)
            return 0
        lax.fori_loop(0, DEG_N_BATCHES, edge_step, 0)
        plsc.subcore_barrier()

        def drain_step(j, _):
            r0 = sid * ROWS_PER_TILE + j * DEG_BATCH
            pltpu.sync_copy(acc_sh.at[pl.ds(r0, DEG_BATCH)], val_v)
            pltpu.sync_copy(val_v, out_hbm.at[cid, pl.ds(r0, DEG_BATCH)])
            return 0
        lax.fori_loop(0, ROWS_PER_TILE // DEG_BATCH, drain_step, 0)

    return degk(dst)


def _grid(n):
    return (n // MM_BLOCK,)


def _tc_dinv(deg2):
    def body(d_ref, o_ref):
        deg = d_ref[0, :] + d_ref[1, :] + 1.0
        o_ref[...] = (1.0 / jnp.sqrt(jnp.maximum(deg, 1.0)))[:, None]

    return pl.pallas_call(
        body,
        grid=_grid(N_PAD),
        in_specs=[pl.BlockSpec((2, MM_BLOCK), lambda i: (0, i))],
        out_specs=pl.BlockSpec((MM_BLOCK, 1), lambda i: (i, 0)),
        out_shape=jax.ShapeDtypeStruct((N_PAD, 1), jnp.float32),
    )(deg2)


def _tc_scale(x, dinv):
    F = x.shape[1]

    def body(x_ref, d_ref, o_ref):
        o_ref[...] = x_ref[...] * d_ref[...]

    return pl.pallas_call(
        body,
        grid=_grid(N_PAD),
        in_specs=[pl.BlockSpec((MM_BLOCK, F), lambda i: (i, 0)),
                  pl.BlockSpec((MM_BLOCK, 1), lambda i: (i, 0))],
        out_specs=pl.BlockSpec((MM_BLOCK, F), lambda i: (i, 0)),
        out_shape=jax.ShapeDtypeStruct((N_PAD, F), jnp.float32),
    )(x, dinv)


def _tc_mm_scale(x, w, dinv):
    fin, fout = w.shape

    def body(x_ref, w_ref, d_ref, o_ref):
        y = jnp.dot(x_ref[...], w_ref[...],
                    preferred_element_type=jnp.float32)
        o_ref[...] = y * d_ref[...]

    return pl.pallas_call(
        body,
        grid=_grid(N_PAD),
        in_specs=[pl.BlockSpec((MM_BLOCK, fin), lambda i: (i, 0)),
                  pl.BlockSpec((fin, fout), lambda i: (0, 0)),
                  pl.BlockSpec((MM_BLOCK, 1), lambda i: (i, 0))],
        out_specs=pl.BlockSpec((MM_BLOCK, fout), lambda i: (i, 0)),
        out_shape=jax.ShapeDtypeStruct((N_PAD, fout), jnp.float32),
    )(x, w, dinv)


def _tc_comb(s, ms, dinv):
    F = s.shape[1]

    def body(s_ref, m_ref, d_ref, o_ref, s1_ref, s2_ref):
        p = (s_ref[...] + m_ref[...]) * d_ref[...]
        o_ref[...] = p

        @pl.when(pl.program_id(0) == 0)
        def _():
            s1_ref[...] = jnp.zeros_like(s1_ref)
            s2_ref[...] = jnp.zeros_like(s2_ref)
        s1_ref[...] += jnp.sum(p, axis=0, keepdims=True)
        s2_ref[...] += jnp.sum(p * p, axis=0, keepdims=True)

    return pl.pallas_call(
        body,
        grid=_grid(N_PAD),
        in_specs=[pl.BlockSpec((MM_BLOCK, F), lambda i: (i, 0)),
                  pl.BlockSpec((MM_BLOCK, F), lambda i: (i, 0)),
                  pl.BlockSpec((MM_BLOCK, 1), lambda i: (i, 0))],
        out_specs=[pl.BlockSpec((MM_BLOCK, F), lambda i: (i, 0)),
                   pl.BlockSpec((1, F), lambda i: (0, 0)),
                   pl.BlockSpec((1, F), lambda i: (0, 0))],
        out_shape=[jax.ShapeDtypeStruct((N_PAD, F), jnp.float32),
                   jax.ShapeDtypeStruct((1, F), jnp.float32),
                   jax.ShapeDtypeStruct((1, F), jnp.float32)],
    )(s, ms, dinv)


def _tc_mm_comb(s, xs, dinv, w):
    fin, fout = w.shape

    def body(s_ref, x_ref, d_ref, w_ref, o_ref, s1_ref, s2_ref):
        g = (s_ref[...] + x_ref[...]) * d_ref[...]
        p = jnp.dot(g, w_ref[...], preferred_element_type=jnp.float32)
        o_ref[...] = p

        @pl.when(pl.program_id(0) == 0)
        def _():
            s1_ref[...] = jnp.zeros_like(s1_ref)
            s2_ref[...] = jnp.zeros_like(s2_ref)
        s1_ref[...] += jnp.sum(p, axis=0, keepdims=True)
        s2_ref[...] += jnp.sum(p * p, axis=0, keepdims=True)

    return pl.pallas_call(
        body,
        grid=_grid(N_PAD),
        in_specs=[pl.BlockSpec((MM_BLOCK, fin), lambda i: (i, 0)),
                  pl.BlockSpec((MM_BLOCK, fin), lambda i: (i, 0)),
                  pl.BlockSpec((MM_BLOCK, 1), lambda i: (i, 0)),
                  pl.BlockSpec((fin, fout), lambda i: (0, 0))],
        out_specs=[pl.BlockSpec((MM_BLOCK, fout), lambda i: (i, 0)),
                   pl.BlockSpec((1, fout), lambda i: (0, 0)),
                   pl.BlockSpec((1, fout), lambda i: (0, 0))],
        out_shape=[jax.ShapeDtypeStruct((N_PAD, fout), jnp.float32),
                   jax.ShapeDtypeStruct((1, fout), jnp.float32),
                   jax.ShapeDtypeStruct((1, fout), jnp.float32)],
    )(s, xs, dinv, w)


def _tc_act(p, s1, s2, gamma, beta, dinv, scale_out):
    F = p.shape[1]

    def body(p_ref, s1_ref, s2_ref, g_ref, b_ref, d_ref, o_ref):
        inv_n = jnp.float32(1.0 / N_REAL)
        mean = s1_ref[...] * inv_n
        var = s2_ref[...] * inv_n - mean * mean
        rstd = 1.0 / jnp.sqrt(var + BN_EPS)
        y = (p_ref[...] - mean) * (rstd * g_ref[...]) + b_ref[...]
        y = jnp.where(y > 0, y, 0.01 * y)
        if scale_out:
            y = y * d_ref[...]
        row = (pl.program_id(0) * MM_BLOCK
               + lax.broadcasted_iota(jnp.int32, (MM_BLOCK, 1), 0))
        o_ref[...] = jnp.where(row < N_REAL, y, 0.0)

    return pl.pallas_call(
        body,
        grid=_grid(N_PAD),
        in_specs=[pl.BlockSpec((MM_BLOCK, F), lambda i: (i, 0)),
                  pl.BlockSpec((1, F), lambda i: (0, 0)),
                  pl.BlockSpec((1, F), lambda i: (0, 0)),
                  pl.BlockSpec((1, F), lambda i: (0, 0)),
                  pl.BlockSpec((1, F), lambda i: (0, 0)),
                  pl.BlockSpec((MM_BLOCK, 1), lambda i: (i, 0))],
        out_specs=pl.BlockSpec((MM_BLOCK, F), lambda i: (i, 0)),
        out_shape=jax.ShapeDtypeStruct((N_PAD, F), jnp.float32),
    )(p, s1, s2, gamma.reshape(1, F), beta.reshape(1, F), dinv)


def _tc_act_mm(p, s1, s2, gamma, beta, w, dinv):
    fin, fout = w.shape

    def body(p_ref, s1_ref, s2_ref, g_ref, b_ref, w_ref, d_ref, o_ref):
        inv_n = jnp.float32(1.0 / N_REAL)
        mean = s1_ref[...] * inv_n
        var = s2_ref[...] * inv_n - mean * mean
        rstd = 1.0 / jnp.sqrt(var + BN_EPS)
        y = (p_ref[...] - mean) * (rstd * g_ref[...]) + b_ref[...]
        y = jnp.where(y > 0, y, 0.01 * y)
        row = (pl.program_id(0) * MM_BLOCK
               + lax.broadcasted_iota(jnp.int32, (MM_BLOCK, 1), 0))
        y = jnp.where(row < N_REAL, y, 0.0)
        o_ref[...] = jnp.dot(y, w_ref[...],
                             preferred_element_type=jnp.float32) * d_ref[...]

    return pl.pallas_call(
        body,
        grid=_grid(N_PAD),
        in_specs=[pl.BlockSpec((MM_BLOCK, fin), lambda i: (i, 0)),
                  pl.BlockSpec((1, fin), lambda i: (0, 0)),
                  pl.BlockSpec((1, fin), lambda i: (0, 0)),
                  pl.BlockSpec((1, fin), lambda i: (0, 0)),
                  pl.BlockSpec((1, fin), lambda i: (0, 0)),
                  pl.BlockSpec((fin, fout), lambda i: (0, 0)),
                  pl.BlockSpec((MM_BLOCK, 1), lambda i: (i, 0))],
        out_specs=pl.BlockSpec((MM_BLOCK, fout), lambda i: (i, 0)),
        out_shape=jax.ShapeDtypeStruct((N_PAD, fout), jnp.float32),
    )(p, s1, s2, gamma.reshape(1, fin), beta.reshape(1, fin), w, dinv)


def _tc_act_head(p, s1, s2, gamma, beta, w1, b1, w2, b2, xpos):
    fin = w1.shape[0]
    fmid = w1.shape[1]
    fout = w2.shape[1]

    def body(p_ref, s1_ref, s2_ref, g_ref, b_ref, w1_ref, b1_ref, w2_ref,
             b2_ref, xp_ref, o_ref):
        inv_n = jnp.float32(1.0 / N_REAL)
        mean = s1_ref[...] * inv_n
        var = s2_ref[...] * inv_n - mean * mean
        rstd = 1.0 / jnp.sqrt(var + BN_EPS)
        y = (p_ref[...] - mean) * (rstd * g_ref[...]) + b_ref[...]
        y = jnp.where(y > 0, y, 0.01 * y)
        h = jnp.dot(y, w1_ref[...],
                    preferred_element_type=jnp.float32) + b1_ref[...]
        h = jnp.where(h > 0, h, 0.01 * h)
        o = jnp.dot(h, w2_ref[...],
                    preferred_element_type=jnp.float32) + b2_ref[...]
        o_ref[...] = o + xp_ref[...]

    return pl.pallas_call(
        body,
        grid=_grid(N_PAD),
        in_specs=[pl.BlockSpec((MM_BLOCK, fin), lambda i: (i, 0)),
                  pl.BlockSpec((1, fin), lambda i: (0, 0)),
                  pl.BlockSpec((1, fin), lambda i: (0, 0)),
                  pl.BlockSpec((1, fin), lambda i: (0, 0)),
                  pl.BlockSpec((1, fin), lambda i: (0, 0)),
                  pl.BlockSpec((fin, fmid), lambda i: (0, 0)),
                  pl.BlockSpec((1, fmid), lambda i: (0, 0)),
                  pl.BlockSpec((fmid, fout), lambda i: (0, 0)),
                  pl.BlockSpec((1, fout), lambda i: (0, 0)),
                  pl.BlockSpec((MM_BLOCK, fout), lambda i: (i, 0))],
        out_specs=pl.BlockSpec((MM_BLOCK, fout), lambda i: (i, 0)),
        out_shape=jax.ShapeDtypeStruct((N_PAD, fout), jnp.float32),
    )(p, s1, s2, gamma.reshape(1, fin), beta.reshape(1, fin), w1,
      b1.reshape(1, fmid), w2, b2.reshape(1, fout), xpos)


def _tc_head(x, w1, b1, w2, b2, xpos):
    fin, fmid = w1.shape
    fout = w2.shape[1]

    def body(x_ref, w1_ref, b1_ref, w2_ref, b2_ref, xp_ref, o_ref):
        h = jnp.dot(x_ref[...], w1_ref[...],
                    preferred_element_type=jnp.float32) + b1_ref[...]
        h = jnp.where(h > 0, h, 0.01 * h)
        o = jnp.dot(h, w2_ref[...],
                    preferred_element_type=jnp.float32) + b2_ref[...]
        o_ref[...] = o + xp_ref[...]

    return pl.pallas_call(
        body,
        grid=_grid(N_PAD),
        in_specs=[pl.BlockSpec((MM_BLOCK, fin), lambda i: (i, 0)),
                  pl.BlockSpec((fin, fmid), lambda i: (0, 0)),
                  pl.BlockSpec((1, fmid), lambda i: (0, 0)),
                  pl.BlockSpec((fmid, fout), lambda i: (0, 0)),
                  pl.BlockSpec((1, fout), lambda i: (0, 0)),
                  pl.BlockSpec((MM_BLOCK, fout), lambda i: (i, 0))],
        out_specs=pl.BlockSpec((MM_BLOCK, fout), lambda i: (i, 0)),
        out_shape=jax.ShapeDtypeStruct((N_PAD, fout), jnp.float32),
    )(x, w1, b1.reshape(1, fmid), w2, b2.reshape(1, fout), xpos)


def kernel(z1, x_pos, params, edge_index):
    n = z1.shape[0]
    src3 = edge_index[0].astype(jnp.int32).reshape(N_SUPER, SUBS, SUB)
    dst = edge_index[1].astype(jnp.int32)
    dst3 = dst.reshape(N_SUPER, SUBS, SUB)
    xpad = jnp.pad(z1, ((0, N_PAD - n), (0, 0)))
    xpos_pad = jnp.pad(x_pos, ((0, N_PAD - n), (0, 0)))

    deg2 = _sc_degree(dst)
    dinv = _tc_dinv(deg2)

    hdims = [16, 32, 64, 128, 256, 256, 512, 512, 256, 256, 128, 64, 32]
    prop_first = [False] * 12

    w0 = params[0][0]
    ms = _tc_mm_scale(xpad, w0, dinv)
    for i in range(12):
        _, _, gamma, beta = params[i]
        s = _from_cm(_sc_propagate(_to_cm(ms), src3, dst3))
        p, s1, s2 = _tc_comb(s, ms, dinv)
        if i < 11:
            w_next = params[i + 1][0]
            ms = _tc_act_mm(p, s1, s2, gamma, beta, w_next, dinv)
        else:
            w1, b1 = params[12]
            w2, b2 = params[13]
            out = _tc_act_head(p, s1, s2, gamma, beta, w1, b1, w2, b2,
                               xpos_pad)
    return out[:n]


# speedup vs baseline: 5.7845x; 1.2483x over previous
import functools

import jax
import jax.numpy as jnp
from jax import lax
from jax.experimental import pallas as pl
from jax.experimental.pallas import tpu as pltpu
from jax.experimental.pallas import tpu_sc as plsc

N_REAL = 100000
N_PAD = 102400
E_TOT = 3200000
N_TILES = 16

SUB = 400
SUBS = 4
SUPER = SUB * SUBS
N_SUPER = E_TOT // SUPER
SUPERS_PER_TILE = N_SUPER // N_TILES
ROWS_PER_TILE = N_PAD // N_TILES

DEG_BATCH = 800
DEG_EDGES_PER_TILE = E_TOT // 32
DEG_N_BATCHES = DEG_EDGES_PER_TILE // DEG_BATCH

BN_EPS = 1e-5
MM_BLOCK = 1024


def _sc_propagate(h_cm, src3, dst3):
    mesh = plsc.VectorSubcoreMesh(core_axis_name="c", subcore_axis_name="s")
    n_chunks = h_cm.shape[0]

    @functools.partial(
        pl.kernel,
        out_type=jax.ShapeDtypeStruct((n_chunks, N_PAD, 16), jnp.float32),
        mesh=mesh,
        compiler_params=pltpu.CompilerParams(use_tc_tiling_on_sc=False),
        scratch_types=[
            pltpu.VMEM_SHARED((N_PAD, 16), jnp.float32),
            pltpu.VMEM((2, SUBS, SUB), jnp.int32),
            pltpu.VMEM((2, SUBS, SUB), jnp.int32),
            pltpu.VMEM((2, SUB, 16), jnp.float32),
            pltpu.SemaphoreType.DMA,
            pltpu.SemaphoreType.DMA,
            pltpu.SemaphoreType.DMA,
            pltpu.SemaphoreType.DMA,
            pltpu.SemaphoreType.DMA,
        ],
    )
    def prop(h_hbm, src_hbm, dst_hbm, out_hbm, acc_sh, src_v, dst_v, rows_v,
             sem_g0, sem_g1, sem_s0, sem_s1, sem_i):
        cid = lax.axis_index("c")
        sid = lax.axis_index("s")
        sems = (sem_s0, sem_s1)
        gsems = (sem_g0, sem_g1)

        def wait_scatter(p):
            pltpu.make_async_copy(
                rows_v.at[p], acc_sh.at[pl.ds(0, SUB)], sems[p]).wait()

        def chunk_body(k):
            def zfill(i, _):
                rows_v[0, i, :] = jnp.zeros((16,), jnp.float32)
                return 0
            lax.fori_loop(0, SUB, zfill, 0)

            def zero_step(j, _):
                r0 = sid * ROWS_PER_TILE + j * SUB
                pltpu.sync_copy(rows_v.at[0], acc_sh.at[pl.ds(r0, SUB)])
                return 0
            lax.fori_loop(0, ROWS_PER_TILE // SUB, zero_step, 0)
            plsc.subcore_barrier()

            def super_step(s, _):
                sp = s % 2
                sv = src_v.at[sp]
                dv = dst_v.at[sp]
                r = sid * SUPERS_PER_TILE + s

                @pl.when(s == 0)
                def _():
                    pltpu.sync_copy(src_hbm.at[r], sv)
                    pltpu.sync_copy(dst_hbm.at[r], dv)

                @pl.when(s > 0)
                def _():
                    pltpu.make_async_copy(src_hbm.at[r], sv, sem_i).wait()
                    pltpu.make_async_copy(dst_hbm.at[r], dv, sem_i).wait()

                def gwait(j):
                    q = j % 2
                    pltpu.make_async_copy(
                        h_hbm.at[k].at[sv.at[j]], rows_v.at[q],
                        gsems[q]).wait()

                def scat(j):
                    q = j % 2
                    pltpu.async_copy(rows_v.at[q], acc_sh.at[dv.at[j]],
                                     sems[q], add=True)

                for j in range(SUBS):
                    p = j % 2

                    @pl.when((s > 0) | (j >= 2))
                    def _():
                        wait_scatter(p)
                    pltpu.async_copy(h_hbm.at[k].at[sv.at[j]],
                                     rows_v.at[p], gsems[p])
                    if j >= 1:
                        gwait(j - 1)
                        scat(j - 1)
                    if j == 1:
                        @pl.when(s < SUPERS_PER_TILE - 1)
                        def _():
                            rn = r + 1
                            pltpu.async_copy(src_hbm.at[rn],
                                             src_v.at[1 - sp], sem_i)
                            pltpu.async_copy(dst_hbm.at[rn],
                                             dst_v.at[1 - sp], sem_i)
                gwait(SUBS - 1)
                scat(SUBS - 1)
                return 0
            lax.fori_loop(0, SUPERS_PER_TILE, super_step, 0)
            wait_scatter(0)
            wait_scatter(1)
            plsc.subcore_barrier()

            r0 = sid * ROWS_PER_TILE
            pltpu.sync_copy(acc_sh.at[pl.ds(r0, ROWS_PER_TILE)],
                            out_hbm.at[k, pl.ds(r0, ROWS_PER_TILE)])
            plsc.subcore_barrier()

        if n_chunks == 1:
            @pl.when(cid == 0)
            def _():
                chunk_body(0)
        else:
            def chunk_step(i, _):
                chunk_body(2 * i + cid)
                return 0
            lax.fori_loop(0, n_chunks // 2, chunk_step, 0)

    return prop(h_cm, src3, dst3)


def _to_cm(h):
    K = h.shape[1] // 16
    if K == 1:
        return h.reshape(1, N_PAD, 16)
    return h.reshape(N_PAD, K, 16).transpose(1, 0, 2)


def _from_cm(s_cm):
    K = s_cm.shape[0]
    if K == 1:
        return s_cm.reshape(N_PAD, 16)
    return s_cm.transpose(1, 0, 2).reshape(N_PAD, K * 16)


def _sc_degree(dst):
    mesh = plsc.VectorSubcoreMesh(core_axis_name="c", subcore_axis_name="s")

    @functools.partial(
        pl.kernel,
        out_type=jax.ShapeDtypeStruct((2, N_PAD), jnp.float32),
        mesh=mesh,
        compiler_params=pltpu.CompilerParams(use_tc_tiling_on_sc=False),
        scratch_types=[
            pltpu.VMEM_SHARED((N_PAD,), jnp.float32),
            pltpu.VMEM((DEG_BATCH,), jnp.int32),
            pltpu.VMEM((DEG_BATCH,), jnp.float32),
        ],
    )
    def degk(dst_hbm, out_hbm, acc_sh, dst_v, val_v):
        cid = lax.axis_index("c")
        sid = lax.axis_index("s")

        def zfill(i, _):
            val_v[pl.ds(16 * i, 16)] = jnp.zeros((16,), jnp.float32)
            return 0
        lax.fori_loop(0, DEG_BATCH // 16, zfill, 0)

        def zero_step(j, _):
            r0 = sid * ROWS_PER_TILE + j * DEG_BATCH
            pltpu.sync_copy(val_v, acc_sh.at[pl.ds(r0, DEG_BATCH)])
            return 0
        lax.fori_loop(0, ROWS_PER_TILE // DEG_BATCH, zero_step, 0)
        plsc.subcore_barrier()

        def ofill(i, _):
            val_v[pl.ds(16 * i, 16)] = jnp.ones((16,), jnp.float32)
            return 0
        lax.fori_loop(0, DEG_BATCH // 16, ofill, 0)

        def edge_step(b, _):
            e0 = (cid * 16 + sid) * DEG_EDGES_PER_TILE + b * DEG_BATCH
            pltpu.sync_copy(dst_hbm.at[pl.ds(e0, DEG_BATCH)], dst_v)
            pltpu.sync_copy(val_v, acc_sh.at[dst_v], add=True)
            return 0
        lax.fori_loop(0, DEG_N_BATCHES, edge_step, 0)
        plsc.subcore_barrier()

        def drain_step(j, _):
            r0 = sid * ROWS_PER_TILE + j * DEG_BATCH
            pltpu.sync_copy(acc_sh.at[pl.ds(r0, DEG_BATCH)], val_v)
            pltpu.sync_copy(val_v, out_hbm.at[cid, pl.ds(r0, DEG_BATCH)])
            return 0
        lax.fori_loop(0, ROWS_PER_TILE // DEG_BATCH, drain_step, 0)

    return degk(dst)


def _grid(n):
    return (n // MM_BLOCK,)


def _tc_dinv(deg2):
    def body(d_ref, o_ref):
        deg = d_ref[0, :] + d_ref[1, :] + 1.0
        o_ref[...] = (1.0 / jnp.sqrt(jnp.maximum(deg, 1.0)))[:, None]

    return pl.pallas_call(
        body,
        grid=_grid(N_PAD),
        in_specs=[pl.BlockSpec((2, MM_BLOCK), lambda i: (0, i))],
        out_specs=pl.BlockSpec((MM_BLOCK, 1), lambda i: (i, 0)),
        out_shape=jax.ShapeDtypeStruct((N_PAD, 1), jnp.float32),
    )(deg2)


def _tc_scale(x, dinv):
    F = x.shape[1]

    def body(x_ref, d_ref, o_ref):
        o_ref[...] = x_ref[...] * d_ref[...]

    return pl.pallas_call(
        body,
        grid=_grid(N_PAD),
        in_specs=[pl.BlockSpec((MM_BLOCK, F), lambda i: (i, 0)),
                  pl.BlockSpec((MM_BLOCK, 1), lambda i: (i, 0))],
        out_specs=pl.BlockSpec((MM_BLOCK, F), lambda i: (i, 0)),
        out_shape=jax.ShapeDtypeStruct((N_PAD, F), jnp.float32),
    )(x, dinv)


def _tc_mm_scale(x, w, dinv):
    fin, fout = w.shape

    def body(x_ref, w_ref, d_ref, o_ref):
        y = jnp.dot(x_ref[...], w_ref[...],
                    preferred_element_type=jnp.float32)
        o_ref[...] = y * d_ref[...]

    return pl.pallas_call(
        body,
        grid=_grid(N_PAD),
        in_specs=[pl.BlockSpec((MM_BLOCK, fin), lambda i: (i, 0)),
                  pl.BlockSpec((fin, fout), lambda i: (0, 0)),
                  pl.BlockSpec((MM_BLOCK, 1), lambda i: (i, 0))],
        out_specs=pl.BlockSpec((MM_BLOCK, fout), lambda i: (i, 0)),
        out_shape=jax.ShapeDtypeStruct((N_PAD, fout), jnp.float32),
    )(x, w, dinv)


def _tc_comb(s, ms, dinv):
    F = s.shape[1]

    def body(s_ref, m_ref, d_ref, o_ref, s1_ref, s2_ref):
        p = (s_ref[...] + m_ref[...]) * d_ref[...]
        o_ref[...] = p

        @pl.when(pl.program_id(0) == 0)
        def _():
            s1_ref[...] = jnp.zeros_like(s1_ref)
            s2_ref[...] = jnp.zeros_like(s2_ref)
        s1_ref[...] += jnp.sum(p, axis=0, keepdims=True)
        s2_ref[...] += jnp.sum(p * p, axis=0, keepdims=True)

    return pl.pallas_call(
        body,
        grid=_grid(N_PAD),
        in_specs=[pl.BlockSpec((MM_BLOCK, F), lambda i: (i, 0)),
                  pl.BlockSpec((MM_BLOCK, F), lambda i: (i, 0)),
                  pl.BlockSpec((MM_BLOCK, 1), lambda i: (i, 0))],
        out_specs=[pl.BlockSpec((MM_BLOCK, F), lambda i: (i, 0)),
                   pl.BlockSpec((1, F), lambda i: (0, 0)),
                   pl.BlockSpec((1, F), lambda i: (0, 0))],
        out_shape=[jax.ShapeDtypeStruct((N_PAD, F), jnp.float32),
                   jax.ShapeDtypeStruct((1, F), jnp.float32),
                   jax.ShapeDtypeStruct((1, F), jnp.float32)],
    )(s, ms, dinv)


def _tc_mm_comb(s, xs, dinv, w):
    fin, fout = w.shape

    def body(s_ref, x_ref, d_ref, w_ref, o_ref, s1_ref, s2_ref):
        g = (s_ref[...] + x_ref[...]) * d_ref[...]
        p = jnp.dot(g, w_ref[...], preferred_element_type=jnp.float32)
        o_ref[...] = p

        @pl.when(pl.program_id(0) == 0)
        def _():
            s1_ref[...] = jnp.zeros_like(s1_ref)
            s2_ref[...] = jnp.zeros_like(s2_ref)
        s1_ref[...] += jnp.sum(p, axis=0, keepdims=True)
        s2_ref[...] += jnp.sum(p * p, axis=0, keepdims=True)

    return pl.pallas_call(
        body,
        grid=_grid(N_PAD),
        in_specs=[pl.BlockSpec((MM_BLOCK, fin), lambda i: (i, 0)),
                  pl.BlockSpec((MM_BLOCK, fin), lambda i: (i, 0)),
                  pl.BlockSpec((MM_BLOCK, 1), lambda i: (i, 0)),
                  pl.BlockSpec((fin, fout), lambda i: (0, 0))],
        out_specs=[pl.BlockSpec((MM_BLOCK, fout), lambda i: (i, 0)),
                   pl.BlockSpec((1, fout), lambda i: (0, 0)),
                   pl.BlockSpec((1, fout), lambda i: (0, 0))],
        out_shape=[jax.ShapeDtypeStruct((N_PAD, fout), jnp.float32),
                   jax.ShapeDtypeStruct((1, fout), jnp.float32),
                   jax.ShapeDtypeStruct((1, fout), jnp.float32)],
    )(s, xs, dinv, w)


def _tc_act(p, s1, s2, gamma, beta, dinv, scale_out):
    F = p.shape[1]

    def body(p_ref, s1_ref, s2_ref, g_ref, b_ref, d_ref, o_ref):
        inv_n = jnp.float32(1.0 / N_REAL)
        mean = s1_ref[...] * inv_n
        var = s2_ref[...] * inv_n - mean * mean
        rstd = 1.0 / jnp.sqrt(var + BN_EPS)
        y = (p_ref[...] - mean) * (rstd * g_ref[...]) + b_ref[...]
        y = jnp.where(y > 0, y, 0.01 * y)
        if scale_out:
            y = y * d_ref[...]
        row = (pl.program_id(0) * MM_BLOCK
               + lax.broadcasted_iota(jnp.int32, (MM_BLOCK, 1), 0))
        o_ref[...] = jnp.where(row < N_REAL, y, 0.0)

    return pl.pallas_call(
        body,
        grid=_grid(N_PAD),
        in_specs=[pl.BlockSpec((MM_BLOCK, F), lambda i: (i, 0)),
                  pl.BlockSpec((1, F), lambda i: (0, 0)),
                  pl.BlockSpec((1, F), lambda i: (0, 0)),
                  pl.BlockSpec((1, F), lambda i: (0, 0)),
                  pl.BlockSpec((1, F), lambda i: (0, 0)),
                  pl.BlockSpec((MM_BLOCK, 1), lambda i: (i, 0))],
        out_specs=pl.BlockSpec((MM_BLOCK, F), lambda i: (i, 0)),
        out_shape=jax.ShapeDtypeStruct((N_PAD, F), jnp.float32),
    )(p, s1, s2, gamma.reshape(1, F), beta.reshape(1, F), dinv)


def _tc_act_mm(p, s1, s2, gamma, beta, w, dinv):
    fin, fout = w.shape

    def body(p_ref, s1_ref, s2_ref, g_ref, b_ref, w_ref, d_ref, o_ref):
        inv_n = jnp.float32(1.0 / N_REAL)
        mean = s1_ref[...] * inv_n
        var = s2_ref[...] * inv_n - mean * mean
        rstd = 1.0 / jnp.sqrt(var + BN_EPS)
        y = (p_ref[...] - mean) * (rstd * g_ref[...]) + b_ref[...]
        y = jnp.where(y > 0, y, 0.01 * y)
        row = (pl.program_id(0) * MM_BLOCK
               + lax.broadcasted_iota(jnp.int32, (MM_BLOCK, 1), 0))
        y = jnp.where(row < N_REAL, y, 0.0)
        o_ref[...] = jnp.dot(y, w_ref[...],
                             preferred_element_type=jnp.float32) * d_ref[...]

    return pl.pallas_call(
        body,
        grid=_grid(N_PAD),
        in_specs=[pl.BlockSpec((MM_BLOCK, fin), lambda i: (i, 0)),
                  pl.BlockSpec((1, fin), lambda i: (0, 0)),
                  pl.BlockSpec((1, fin), lambda i: (0, 0)),
                  pl.BlockSpec((1, fin), lambda i: (0, 0)),
                  pl.BlockSpec((1, fin), lambda i: (0, 0)),
                  pl.BlockSpec((fin, fout), lambda i: (0, 0)),
                  pl.BlockSpec((MM_BLOCK, 1), lambda i: (i, 0))],
        out_specs=pl.BlockSpec((MM_BLOCK, fout), lambda i: (i, 0)),
        out_shape=jax.ShapeDtypeStruct((N_PAD, fout), jnp.float32),
    )(p, s1, s2, gamma.reshape(1, fin), beta.reshape(1, fin), w, dinv)


def _tc_act_head(p, s1, s2, gamma, beta, w1, b1, w2, b2, xpos):
    fin = w1.shape[0]
    fmid = w1.shape[1]
    fout = w2.shape[1]

    def body(p_ref, s1_ref, s2_ref, g_ref, b_ref, w1_ref, b1_ref, w2_ref,
             b2_ref, xp_ref, o_ref):
        inv_n = jnp.float32(1.0 / N_REAL)
        mean = s1_ref[...] * inv_n
        var = s2_ref[...] * inv_n - mean * mean
        rstd = 1.0 / jnp.sqrt(var + BN_EPS)
        y = (p_ref[...] - mean) * (rstd * g_ref[...]) + b_ref[...]
        y = jnp.where(y > 0, y, 0.01 * y)
        h = jnp.dot(y, w1_ref[...],
                    preferred_element_type=jnp.float32) + b1_ref[...]
        h = jnp.where(h > 0, h, 0.01 * h)
        o = jnp.dot(h, w2_ref[...],
                    preferred_element_type=jnp.float32) + b2_ref[...]
        o_ref[...] = o + xp_ref[...]

    return pl.pallas_call(
        body,
        grid=_grid(N_PAD),
        in_specs=[pl.BlockSpec((MM_BLOCK, fin), lambda i: (i, 0)),
                  pl.BlockSpec((1, fin), lambda i: (0, 0)),
                  pl.BlockSpec((1, fin), lambda i: (0, 0)),
                  pl.BlockSpec((1, fin), lambda i: (0, 0)),
                  pl.BlockSpec((1, fin), lambda i: (0, 0)),
                  pl.BlockSpec((fin, fmid), lambda i: (0, 0)),
                  pl.BlockSpec((1, fmid), lambda i: (0, 0)),
                  pl.BlockSpec((fmid, fout), lambda i: (0, 0)),
                  pl.BlockSpec((1, fout), lambda i: (0, 0)),
                  pl.BlockSpec((MM_BLOCK, fout), lambda i: (i, 0))],
        out_specs=pl.BlockSpec((MM_BLOCK, fout), lambda i: (i, 0)),
        out_shape=jax.ShapeDtypeStruct((N_PAD, fout), jnp.float32),
    )(p, s1, s2, gamma.reshape(1, fin), beta.reshape(1, fin), w1,
      b1.reshape(1, fmid), w2, b2.reshape(1, fout), xpos)


def _tc_head(x, w1, b1, w2, b2, xpos):
    fin, fmid = w1.shape
    fout = w2.shape[1]

    def body(x_ref, w1_ref, b1_ref, w2_ref, b2_ref, xp_ref, o_ref):
        h = jnp.dot(x_ref[...], w1_ref[...],
                    preferred_element_type=jnp.float32) + b1_ref[...]
        h = jnp.where(h > 0, h, 0.01 * h)
        o = jnp.dot(h, w2_ref[...],
                    preferred_element_type=jnp.float32) + b2_ref[...]
        o_ref[...] = o + xp_ref[...]

    return pl.pallas_call(
        body,
        grid=_grid(N_PAD),
        in_specs=[pl.BlockSpec((MM_BLOCK, fin), lambda i: (i, 0)),
                  pl.BlockSpec((fin, fmid), lambda i: (0, 0)),
                  pl.BlockSpec((1, fmid), lambda i: (0, 0)),
                  pl.BlockSpec((fmid, fout), lambda i: (0, 0)),
                  pl.BlockSpec((1, fout), lambda i: (0, 0)),
                  pl.BlockSpec((MM_BLOCK, fout), lambda i: (i, 0))],
        out_specs=pl.BlockSpec((MM_BLOCK, fout), lambda i: (i, 0)),
        out_shape=jax.ShapeDtypeStruct((N_PAD, fout), jnp.float32),
    )(x, w1, b1.reshape(1, fmid), w2, b2.reshape(1, fout), xpos)


def kernel(z1, x_pos, params, edge_index):
    n = z1.shape[0]
    src3 = edge_index[0].astype(jnp.int32).reshape(N_SUPER, SUBS, SUB)
    dst = edge_index[1].astype(jnp.int32)
    dst3 = dst.reshape(N_SUPER, SUBS, SUB)
    xpad = jnp.pad(z1, ((0, N_PAD - n), (0, 0)))
    xpos_pad = jnp.pad(x_pos, ((0, N_PAD - n), (0, 0)))

    deg2 = _sc_degree(dst)
    dinv = _tc_dinv(deg2)

    hdims = [16, 32, 64, 128, 256, 256, 512, 512, 256, 256, 128, 64, 32]
    prop_first = [False] * 12

    w0 = params[0][0]
    ms = _tc_mm_scale(xpad, w0, dinv)
    for i in range(12):
        _, _, gamma, beta = params[i]
        s = _from_cm(_sc_propagate(_to_cm(ms), src3, dst3))
        p, s1, s2 = _tc_comb(s, ms, dinv)
        if i < 11:
            w_next = params[i + 1][0]
            ms = _tc_act_mm(p, s1, s2, gamma, beta, w_next, dinv)
        else:
            w1, b1 = params[12]
            w2, b2 = params[13]
            out = _tc_act_head(p, s1, s2, gamma, beta, w1, b1, w2, b2,
                               xpos_pad)
    return out[:n]
